# baseline (speedup 1.0000x reference)
import functools

import jax
import jax.numpy as jnp
from jax import lax
from jax.experimental import pallas as pl
from jax.experimental.pallas import tpu as pltpu
from jax.experimental.pallas import tpu_sc as plsc

VOCAB = 1000000
EMBED_DIM = 64
BATCH = 4096
HIST = 200

NUM_SC = 2
NUM_SUBCORES = 16
NUM_WORKERS = NUM_SC * NUM_SUBCORES
LANES = 16

CHUNK = 800
NCHUNKS = VOCAB // CHUNK
CHUNKS_BASE = NCHUNKS // NUM_WORKERS
CHUNKS_EXTRA = NCHUNKS % NUM_WORKERS
GROUPS_C = CHUNK // LANES
STRIP = 10

ROWS_PER_W = BATCH // NUM_WORKERS
TOK_PER_W = ROWS_PER_W * HIST
GROUPS_P = ROWS_PER_W // LANES


def _wid():
    return lax.axis_index("c") * NUM_SUBCORES + lax.axis_index("s")


def _matvec_body(table_hbm, aux_hbm, scores_hbm, buf_v, acc_v, wb_v, aux_v,
                 coltab_v, sem):
    wid = _wid()
    iota = lax.iota(jnp.int32, LANES)

    pltpu.sync_copy(aux_hbm, aux_v)
    for d in range(EMBED_DIM):
        coltab_v[pl.ds(d * LANES, LANES)] = jnp.full((LANES,), d, jnp.int32)
        wb_v[pl.ds(d * LANES, LANES)] = plsc.load_gather(
            aux_v, [jnp.full((LANES,), d + 1, jnp.int32)])
    bias_s = plsc.load_gather(
        aux_v, [jnp.full((LANES,), EMBED_DIM + 1, jnp.int32)]) * (1.0 / HIST)

    start = wid * CHUNKS_BASE + jnp.minimum(wid, CHUNKS_EXTRA)
    cnt = CHUNKS_BASE + (wid < CHUNKS_EXTRA).astype(jnp.int32)

    def chunk_body(k, carry):
        c = start + k
        pltpu.sync_copy(table_hbm.at[pl.ds(c * CHUNK, CHUNK)], buf_v)

        for strip in range(GROUPS_C // STRIP):
            def d_body(d, accs):
                wb = wb_v[pl.ds(d * LANES, LANES)]
                col = coltab_v[pl.ds(d * LANES, LANES)]
                return tuple(
                    accs[j] + plsc.load_gather(
                        buf_v, [iota + (strip * STRIP + j) * LANES, col]) * wb
                    for j in range(STRIP)
                )

            init = tuple(jnp.zeros((LANES,), jnp.float32)
                         for _ in range(STRIP))
            accs = lax.fori_loop(0, EMBED_DIM, d_body, init)
            for j in range(STRIP):
                g = strip * STRIP + j
                acc_v[pl.ds(g * LANES, LANES)] = (
                    accs[j] * (1.0 / HIST) + bias_s)

        pltpu.sync_copy(acc_v, scores_hbm.at[pl.ds(c * CHUNK, CHUNK)])
        return carry

    lax.fori_loop(0, cnt, chunk_body, 0)


def _pool_body(xflat_hbm, scores_hbm, out_hbm, idx_v, vals_v, res_v,
               ltab_v, sem):
    wid = _wid()
    iota = lax.iota(jnp.int32, LANES)
    base = wid * TOK_PER_W

    for l in range(HIST):
        ltab_v[pl.ds(l * LANES, LANES)] = jnp.full((LANES,), l, jnp.int32)

    pltpu.sync_copy(xflat_hbm.at[pl.ds(base, TOK_PER_W)], idx_v)
    pltpu.async_copy(scores_hbm.at[idx_v], vals_v, sem).wait()

    bases = [iota * HIST + (g * LANES * HIST) for g in range(GROUPS_P)]

    def body(l, accs):
        lvec = ltab_v[pl.ds(l * LANES, LANES)]
        return tuple(
            accs[g] + plsc.load_gather(vals_v, [bases[g] + lvec])
            for g in range(GROUPS_P)
        )

    init = tuple(jnp.zeros((LANES,), jnp.float32) for _ in range(GROUPS_P))
    accs = lax.fori_loop(0, HIST, body, init)

    for g in range(GROUPS_P):
        z = accs[g]
        res_v[pl.ds(g * LANES, LANES)] = 1.0 / (1.0 + jnp.exp(-z))
    pltpu.sync_copy(res_v, out_hbm.at[pl.ds(wid * ROWS_PER_W, ROWS_PER_W)])


@functools.cache
def _make_kernels():
    mesh = plsc.VectorSubcoreMesh(
        core_axis_name="c", subcore_axis_name="s",
        num_cores=NUM_SC, num_subcores=NUM_SUBCORES)
    params = pltpu.CompilerParams(needs_layout_passes=False)
    matvec = pl.kernel(
        _matvec_body,
        out_type=jax.ShapeDtypeStruct((VOCAB,), jnp.float32),
        mesh=mesh,
        compiler_params=params,
        scratch_types=[
            pltpu.VMEM((CHUNK, EMBED_DIM), jnp.float32),
            pltpu.VMEM((CHUNK,), jnp.float32),
            pltpu.VMEM((EMBED_DIM * LANES,), jnp.float32),
            pltpu.VMEM((80,), jnp.float32),
            pltpu.VMEM((EMBED_DIM * LANES,), jnp.int32),
            pltpu.SemaphoreType.DMA,
        ],
    )
    pool = pl.kernel(
        _pool_body,
        out_type=jax.ShapeDtypeStruct((BATCH,), jnp.float32),
        mesh=mesh,
        compiler_params=params,
        scratch_types=[
            pltpu.VMEM((TOK_PER_W,), jnp.int32),
            pltpu.VMEM((TOK_PER_W,), jnp.float32),
            pltpu.VMEM((ROWS_PER_W,), jnp.float32),
            pltpu.VMEM((HIST * LANES,), jnp.int32),
            pltpu.SemaphoreType.DMA,
        ],
    )
    return matvec, pool


def _tc_scores_body(table_ref, w_ref, b_ref, out_ref):
    t = table_ref[...]
    w = w_ref[...]
    s = jnp.sum(t * w, axis=1, keepdims=True)
    out_ref[...] = (s + b_ref[0, 0]) * (1.0 / HIST)


def _tc_scores(table, fc_w, fc_b):
    RB = 25000
    return pl.pallas_call(
        _tc_scores_body,
        grid=(VOCAB // RB,),
        in_specs=[
            pl.BlockSpec((RB, EMBED_DIM), lambda i: (i, 0)),
            pl.BlockSpec((1, EMBED_DIM), lambda i: (0, 0)),
            pl.BlockSpec(memory_space=pltpu.SMEM),
        ],
        out_specs=pl.BlockSpec((RB, 1), lambda i: (i, 0)),
        out_shape=jax.ShapeDtypeStruct((VOCAB, 1), jnp.float32),
    )(table, fc_w, fc_b.reshape(1, 1))


def kernel(x, table, fc_w, fc_b):
    matvec, pool = _make_kernels()
    aux = jnp.concatenate(
        [jnp.zeros((1,), jnp.float32), fc_w.reshape(-1), fc_b.reshape(-1),
         jnp.zeros((14,), jnp.float32)])
    scores = matvec(table, aux)
    out = pool(x.reshape(-1), scores)
    return out.reshape(BATCH, 1)


# speedup vs baseline: 1.0240x; 1.0240x over previous
import functools

import jax
import jax.numpy as jnp
from jax import lax
from jax.experimental import pallas as pl
from jax.experimental.pallas import tpu as pltpu
from jax.experimental.pallas import tpu_sc as plsc

VOCAB = 1000000
EMBED_DIM = 64
BATCH = 4096
HIST = 200

NUM_SC = 2
NUM_SUBCORES = 16
NUM_WORKERS = NUM_SC * NUM_SUBCORES
LANES = 16

CHUNK = 800
NCHUNKS = VOCAB // CHUNK
CHUNKS_BASE = NCHUNKS // NUM_WORKERS
CHUNKS_EXTRA = NCHUNKS % NUM_WORKERS
GROUPS_C = CHUNK // LANES
STRIP = 10

ROWS_PER_W = BATCH // NUM_WORKERS
TOK_PER_W = ROWS_PER_W * HIST
GROUPS_P = ROWS_PER_W // LANES


def _wid():
    return lax.axis_index("c") * NUM_SUBCORES + lax.axis_index("s")


def _matvec_body(table_hbm, aux_hbm, scores_hbm, buf_v, acc_v, wb_v, aux_v,
                 coltab_v, sem):
    wid = _wid()
    iota = lax.iota(jnp.int32, LANES)

    pltpu.sync_copy(aux_hbm, aux_v)
    for d in range(EMBED_DIM):
        coltab_v[pl.ds(d * LANES, LANES)] = jnp.full((LANES,), d, jnp.int32)
        wb_v[pl.ds(d * LANES, LANES)] = plsc.load_gather(
            aux_v, [jnp.full((LANES,), d + 1, jnp.int32)])
    bias_s = plsc.load_gather(
        aux_v, [jnp.full((LANES,), EMBED_DIM + 1, jnp.int32)]) * (1.0 / HIST)

    start = wid * CHUNKS_BASE + jnp.minimum(wid, CHUNKS_EXTRA)
    cnt = CHUNKS_BASE + (wid < CHUNKS_EXTRA).astype(jnp.int32)

    def chunk_body(k, carry):
        c = start + k
        pltpu.sync_copy(table_hbm.at[pl.ds(c * CHUNK, CHUNK)], buf_v)

        for strip in range(GROUPS_C // STRIP):
            def d_body(d, accs, _strip=strip):
                wb = wb_v[pl.ds(d * LANES, LANES)]
                col = coltab_v[pl.ds(d * LANES, LANES)]
                return tuple(
                    accs[j] + plsc.load_gather(
                        buf_v, [iota + (_strip * STRIP + j) * LANES, col]) * wb
                    for j in range(STRIP)
                )

            init = tuple(jnp.zeros((LANES,), jnp.float32)
                         for _ in range(STRIP))
            accs = plsc.parallel_loop(
                0, EMBED_DIM, carry=init, unroll=8)(d_body)
            for j in range(STRIP):
                g = strip * STRIP + j
                acc_v[pl.ds(g * LANES, LANES)] = (
                    accs[j] * (1.0 / HIST) + bias_s)

        pltpu.sync_copy(acc_v, scores_hbm.at[pl.ds(c * CHUNK, CHUNK)])
        return carry

    lax.fori_loop(0, cnt, chunk_body, 0)


def _pool_body(xflat_hbm, scores_hbm, out_hbm, idx_v, vals_v, res_v,
               ltab_v, sem):
    wid = _wid()
    iota = lax.iota(jnp.int32, LANES)
    base = wid * TOK_PER_W

    for l in range(HIST):
        ltab_v[pl.ds(l * LANES, LANES)] = jnp.full((LANES,), l, jnp.int32)

    pltpu.sync_copy(xflat_hbm.at[pl.ds(base, TOK_PER_W)], idx_v)
    pltpu.async_copy(scores_hbm.at[idx_v], vals_v, sem).wait()

    bases = [iota * HIST + (g * LANES * HIST) for g in range(GROUPS_P)]

    def body(l, accs):
        lvec = ltab_v[pl.ds(l * LANES, LANES)]
        return tuple(
            accs[g] + plsc.load_gather(vals_v, [bases[g] + lvec])
            for g in range(GROUPS_P)
        )

    init = tuple(jnp.zeros((LANES,), jnp.float32) for _ in range(GROUPS_P))
    accs = lax.fori_loop(0, HIST, body, init)

    for g in range(GROUPS_P):
        z = accs[g]
        res_v[pl.ds(g * LANES, LANES)] = 1.0 / (1.0 + jnp.exp(-z))
    pltpu.sync_copy(res_v, out_hbm.at[pl.ds(wid * ROWS_PER_W, ROWS_PER_W)])


@functools.cache
def _make_kernels():
    mesh = plsc.VectorSubcoreMesh(
        core_axis_name="c", subcore_axis_name="s",
        num_cores=NUM_SC, num_subcores=NUM_SUBCORES)
    params = pltpu.CompilerParams(needs_layout_passes=False)
    matvec = pl.kernel(
        _matvec_body,
        out_type=jax.ShapeDtypeStruct((VOCAB,), jnp.float32),
        mesh=mesh,
        compiler_params=params,
        scratch_types=[
            pltpu.VMEM((CHUNK, EMBED_DIM), jnp.float32),
            pltpu.VMEM((CHUNK,), jnp.float32),
            pltpu.VMEM((EMBED_DIM * LANES,), jnp.float32),
            pltpu.VMEM((80,), jnp.float32),
            pltpu.VMEM((EMBED_DIM * LANES,), jnp.int32),
            pltpu.SemaphoreType.DMA,
        ],
    )
    pool = pl.kernel(
        _pool_body,
        out_type=jax.ShapeDtypeStruct((BATCH,), jnp.float32),
        mesh=mesh,
        compiler_params=params,
        scratch_types=[
            pltpu.VMEM((TOK_PER_W,), jnp.int32),
            pltpu.VMEM((TOK_PER_W,), jnp.float32),
            pltpu.VMEM((ROWS_PER_W,), jnp.float32),
            pltpu.VMEM((HIST * LANES,), jnp.int32),
            pltpu.SemaphoreType.DMA,
        ],
    )
    return matvec, pool


def _tc_scores_body(table_ref, w_ref, b_ref, out_ref):
    t = table_ref[...]
    w = w_ref[...]
    s = jnp.sum(t * w, axis=1, keepdims=True)
    out_ref[...] = (s + b_ref[0, 0]) * (1.0 / HIST)


def _tc_scores(table, fc_w, fc_b):
    RB = 25000
    return pl.pallas_call(
        _tc_scores_body,
        grid=(VOCAB // RB,),
        in_specs=[
            pl.BlockSpec((RB, EMBED_DIM), lambda i: (i, 0)),
            pl.BlockSpec((1, EMBED_DIM), lambda i: (0, 0)),
            pl.BlockSpec(memory_space=pltpu.SMEM),
        ],
        out_specs=pl.BlockSpec((RB, 1), lambda i: (i, 0)),
        out_shape=jax.ShapeDtypeStruct((VOCAB, 1), jnp.float32),
    )(table, fc_w, fc_b.reshape(1, 1))


def kernel(x, table, fc_w, fc_b):
    matvec, pool = _make_kernels()
    aux = jnp.concatenate(
        [jnp.zeros((1,), jnp.float32), fc_w.reshape(-1), fc_b.reshape(-1),
         jnp.zeros((14,), jnp.float32)])
    scores = matvec(table, aux)
    out = pool(x.reshape(-1), scores)
    return out.reshape(BATCH, 1)


# speedup vs baseline: 1.1884x; 1.1605x over previous
import functools

import jax
import jax.numpy as jnp
from jax import lax
from jax.experimental import pallas as pl
from jax.experimental.pallas import tpu as pltpu
from jax.experimental.pallas import tpu_sc as plsc

VOCAB = 1000000
EMBED_DIM = 64
BATCH = 4096
HIST = 200

NUM_SC = 2
NUM_SUBCORES = 16
NUM_WORKERS = NUM_SC * NUM_SUBCORES
LANES = 16

CHUNK = 400
NCHUNKS = VOCAB // CHUNK
CHUNKS_BASE = NCHUNKS // NUM_WORKERS
CHUNKS_EXTRA = NCHUNKS % NUM_WORKERS
GROUPS_C = CHUNK // LANES
STRIP = 5

ROWS_PER_W = BATCH // NUM_WORKERS
TOK_PER_W = ROWS_PER_W * HIST
GROUPS_P = ROWS_PER_W // LANES


def _wid():
    return lax.axis_index("c") * NUM_SUBCORES + lax.axis_index("s")


def _matvec_body(table_hbm, aux_hbm, scores_hbm, buf0_v, buf1_v, acc_v, wb_v,
                 aux_v, coltab_v, sem0, sem1):
    wid = _wid()
    iota = lax.iota(jnp.int32, LANES)

    pltpu.sync_copy(aux_hbm, aux_v)
    for d in range(EMBED_DIM):
        coltab_v[pl.ds(d * LANES, LANES)] = jnp.full((LANES,), d, jnp.int32)
        wb_v[pl.ds(d * LANES, LANES)] = plsc.load_gather(
            aux_v, [jnp.full((LANES,), d + 1, jnp.int32)])
    bias_s = plsc.load_gather(
        aux_v, [jnp.full((LANES,), EMBED_DIM + 1, jnp.int32)]) * (1.0 / HIST)

    start = wid * CHUNKS_BASE + jnp.minimum(wid, CHUNKS_EXTRA)
    cnt = CHUNKS_BASE + (wid < CHUNKS_EXTRA).astype(jnp.int32)

    bufs = (buf0_v, buf1_v)
    sems = (sem0, sem1)

    def _compute(buf, c):
        for strip in range(GROUPS_C // STRIP):
            def d_body(d, accs, _strip=strip):
                wb = wb_v[pl.ds(d * LANES, LANES)]
                col = coltab_v[pl.ds(d * LANES, LANES)]
                return tuple(
                    accs[j] + plsc.load_gather(
                        buf, [iota + (_strip * STRIP + j) * LANES, col]) * wb
                    for j in range(STRIP)
                )

            init = tuple(jnp.zeros((LANES,), jnp.float32)
                         for _ in range(STRIP))
            accs = plsc.parallel_loop(
                0, EMBED_DIM, carry=init, unroll=2)(d_body)
            for j in range(STRIP):
                g = strip * STRIP + j
                acc_v[pl.ds(g * LANES, LANES)] = (
                    accs[j] * (1.0 / HIST) + bias_s)
        pltpu.sync_copy(acc_v, scores_hbm.at[pl.ds(c * CHUNK, CHUNK)])

    pltpu.async_copy(
        table_hbm.at[pl.ds(start * CHUNK, CHUNK)], buf0_v, sem0)

    def pair_body(p, carry):
        for par in range(2):
            k = 2 * p + par
            c = start + k
            buf, sem = bufs[par], sems[par]
            obuf, osem = bufs[1 - par], sems[1 - par]

            @pl.when(k < cnt)
            def _():
                pltpu.make_async_copy(
                    table_hbm.at[pl.ds(c * CHUNK, CHUNK)], buf, sem).wait()

                @pl.when(k + 1 < cnt)
                def _():
                    pltpu.async_copy(
                        table_hbm.at[pl.ds((c + 1) * CHUNK, CHUNK)],
                        obuf, osem)

                _compute(buf, c)
        return carry

    lax.fori_loop(0, (CHUNKS_BASE + 2) // 2, pair_body, 0)


def _pool_body(xflat_hbm, scores_hbm, out_hbm, idx_v, vals_v, res_v,
               ltab_v, sem):
    wid = _wid()
    iota = lax.iota(jnp.int32, LANES)
    base = wid * TOK_PER_W

    for l in range(HIST):
        ltab_v[pl.ds(l * LANES, LANES)] = jnp.full((LANES,), l, jnp.int32)

    pltpu.sync_copy(xflat_hbm.at[pl.ds(base, TOK_PER_W)], idx_v)
    pltpu.async_copy(scores_hbm.at[idx_v], vals_v, sem).wait()

    bases = [iota * HIST + (g * LANES * HIST) for g in range(GROUPS_P)]

    def body(l, accs):
        lvec = ltab_v[pl.ds(l * LANES, LANES)]
        return tuple(
            accs[g] + plsc.load_gather(vals_v, [bases[g] + lvec])
            for g in range(GROUPS_P)
        )

    init = tuple(jnp.zeros((LANES,), jnp.float32) for _ in range(GROUPS_P))
    accs = lax.fori_loop(0, HIST, body, init)

    for g in range(GROUPS_P):
        z = accs[g]
        res_v[pl.ds(g * LANES, LANES)] = 1.0 / (1.0 + jnp.exp(-z))
    pltpu.sync_copy(res_v, out_hbm.at[pl.ds(wid * ROWS_PER_W, ROWS_PER_W)])


@functools.cache
def _make_kernels():
    mesh = plsc.VectorSubcoreMesh(
        core_axis_name="c", subcore_axis_name="s",
        num_cores=NUM_SC, num_subcores=NUM_SUBCORES)
    params = pltpu.CompilerParams(needs_layout_passes=False)
    matvec = pl.kernel(
        _matvec_body,
        out_type=jax.ShapeDtypeStruct((VOCAB,), jnp.float32),
        mesh=mesh,
        compiler_params=params,
        scratch_types=[
            pltpu.VMEM((CHUNK, EMBED_DIM), jnp.float32),
            pltpu.VMEM((CHUNK, EMBED_DIM), jnp.float32),
            pltpu.VMEM((CHUNK,), jnp.float32),
            pltpu.VMEM((EMBED_DIM * LANES,), jnp.float32),
            pltpu.VMEM((80,), jnp.float32),
            pltpu.VMEM((EMBED_DIM * LANES,), jnp.int32),
            pltpu.SemaphoreType.DMA,
            pltpu.SemaphoreType.DMA,
        ],
    )
    pool = pl.kernel(
        _pool_body,
        out_type=jax.ShapeDtypeStruct((BATCH,), jnp.float32),
        mesh=mesh,
        compiler_params=params,
        scratch_types=[
            pltpu.VMEM((TOK_PER_W,), jnp.int32),
            pltpu.VMEM((TOK_PER_W,), jnp.float32),
            pltpu.VMEM((ROWS_PER_W,), jnp.float32),
            pltpu.VMEM((HIST * LANES,), jnp.int32),
            pltpu.SemaphoreType.DMA,
        ],
    )
    return matvec, pool


def _tc_scores_body(table_ref, w_ref, b_ref, out_ref):
    t = table_ref[...]
    w = w_ref[...]
    s = jnp.sum(t * w, axis=1, keepdims=True)
    out_ref[...] = (s + b_ref[0, 0]) * (1.0 / HIST)


def _tc_scores(table, fc_w, fc_b):
    RB = 25000
    return pl.pallas_call(
        _tc_scores_body,
        grid=(VOCAB // RB,),
        in_specs=[
            pl.BlockSpec((RB, EMBED_DIM), lambda i: (i, 0)),
            pl.BlockSpec((1, EMBED_DIM), lambda i: (0, 0)),
            pl.BlockSpec(memory_space=pltpu.SMEM),
        ],
        out_specs=pl.BlockSpec((RB, 1), lambda i: (i, 0)),
        out_shape=jax.ShapeDtypeStruct((VOCAB, 1), jnp.float32),
    )(table, fc_w, fc_b.reshape(1, 1))


def kernel(x, table, fc_w, fc_b):
    matvec, pool = _make_kernels()
    aux = jnp.concatenate(
        [jnp.zeros((1,), jnp.float32), fc_w.reshape(-1), fc_b.reshape(-1),
         jnp.zeros((14,), jnp.float32)])
    scores = matvec(table, aux)
    out = pool(x.reshape(-1), scores)
    return out.reshape(BATCH, 1)


# speedup vs baseline: 2.2129x; 1.8622x over previous
import functools

import jax
import jax.numpy as jnp
from jax import lax
from jax.experimental import pallas as pl
from jax.experimental.pallas import tpu as pltpu
from jax.experimental.pallas import tpu_sc as plsc

VOCAB = 1000000
EMBED_DIM = 64
BATCH = 4096
HIST = 200

NUM_SC = 2
NUM_SUBCORES = 16
NUM_WORKERS = NUM_SC * NUM_SUBCORES
LANES = 16

TC_ROWS = 650000
TC_BLOCK = 25000
SC_ROWS = VOCAB - TC_ROWS
CHUNK = 400
NCHUNKS = SC_ROWS // CHUNK
CHUNKS_BASE = NCHUNKS // NUM_WORKERS
CHUNKS_EXTRA = NCHUNKS % NUM_WORKERS
GROUPS_C = CHUNK // LANES
STRIP = 5

ROWS_PER_W = BATCH // NUM_WORKERS
TOK_PER_W = ROWS_PER_W * HIST
GROUPS_P = ROWS_PER_W // LANES


def _wid():
    return lax.axis_index("c") * NUM_SUBCORES + lax.axis_index("s")


def _matvec_body(table_hbm, aux_hbm, scores_hbm, buf0_v, buf1_v, acc_v, wb_v,
                 aux_v, coltab_v, sem0, sem1):
    wid = _wid()
    iota = lax.iota(jnp.int32, LANES)

    pltpu.sync_copy(aux_hbm, aux_v)
    for d in range(EMBED_DIM):
        coltab_v[pl.ds(d * LANES, LANES)] = jnp.full((LANES,), d, jnp.int32)
        wb_v[pl.ds(d * LANES, LANES)] = plsc.load_gather(
            aux_v, [jnp.full((LANES,), d + 1, jnp.int32)])
    bias_s = plsc.load_gather(
        aux_v, [jnp.full((LANES,), EMBED_DIM + 1, jnp.int32)]) * (1.0 / HIST)

    start = wid * CHUNKS_BASE + jnp.minimum(wid, CHUNKS_EXTRA)
    cnt = CHUNKS_BASE + (wid < CHUNKS_EXTRA).astype(jnp.int32)

    bufs = (buf0_v, buf1_v)
    sems = (sem0, sem1)

    def _compute(buf, c):
        for strip in range(GROUPS_C // STRIP):
            def d_body(d, accs, _strip=strip):
                wb = wb_v[pl.ds(d * LANES, LANES)]
                col = coltab_v[pl.ds(d * LANES, LANES)]
                return tuple(
                    accs[j] + plsc.load_gather(
                        buf, [iota + (_strip * STRIP + j) * LANES, col]) * wb
                    for j in range(STRIP)
                )

            init = tuple(jnp.zeros((LANES,), jnp.float32)
                         for _ in range(STRIP))
            accs = plsc.parallel_loop(
                0, EMBED_DIM, carry=init, unroll=2)(d_body)
            for j in range(STRIP):
                g = strip * STRIP + j
                acc_v[pl.ds(g * LANES, LANES)] = (
                    accs[j] * (1.0 / HIST) + bias_s)
        pltpu.sync_copy(acc_v, scores_hbm.at[pl.ds(c * CHUNK, CHUNK)])

    pltpu.async_copy(
        table_hbm.at[pl.ds(TC_ROWS + start * CHUNK, CHUNK)], buf0_v, sem0)

    def pair_body(p, carry):
        for par in range(2):
            k = 2 * p + par
            c = start + k
            buf, sem = bufs[par], sems[par]
            obuf, osem = bufs[1 - par], sems[1 - par]

            @pl.when(k < cnt)
            def _():
                pltpu.make_async_copy(
                    table_hbm.at[pl.ds(TC_ROWS + c * CHUNK, CHUNK)],
                    buf, sem).wait()

                @pl.when(k + 1 < cnt)
                def _():
                    pltpu.async_copy(
                        table_hbm.at[pl.ds(TC_ROWS + (c + 1) * CHUNK, CHUNK)],
                        obuf, osem)

                _compute(buf, c)
        return carry

    lax.fori_loop(0, (CHUNKS_BASE + 2) // 2, pair_body, 0)


def _pool_body(xflat_hbm, scores_hbm, out_hbm, idx_v, vals_v, res_v,
               ltab_v, sem):
    wid = _wid()
    iota = lax.iota(jnp.int32, LANES)
    base = wid * TOK_PER_W

    for l in range(HIST):
        ltab_v[pl.ds(l * LANES, LANES)] = jnp.full((LANES,), l, jnp.int32)

    pltpu.sync_copy(xflat_hbm.at[pl.ds(base, TOK_PER_W)], idx_v)
    pltpu.async_copy(scores_hbm.at[idx_v], vals_v, sem).wait()

    bases = [iota * HIST + (g * LANES * HIST) for g in range(GROUPS_P)]

    def body(l, accs):
        lvec = ltab_v[pl.ds(l * LANES, LANES)]
        return tuple(
            accs[g] + plsc.load_gather(vals_v, [bases[g] + lvec])
            for g in range(GROUPS_P)
        )

    init = tuple(jnp.zeros((LANES,), jnp.float32) for _ in range(GROUPS_P))
    accs = lax.fori_loop(0, HIST, body, init)

    for g in range(GROUPS_P):
        z = accs[g]
        res_v[pl.ds(g * LANES, LANES)] = 1.0 / (1.0 + jnp.exp(-z))
    pltpu.sync_copy(res_v, out_hbm.at[pl.ds(wid * ROWS_PER_W, ROWS_PER_W)])


@functools.cache
def _make_kernels():
    mesh = plsc.VectorSubcoreMesh(
        core_axis_name="c", subcore_axis_name="s",
        num_cores=NUM_SC, num_subcores=NUM_SUBCORES)
    params = pltpu.CompilerParams(needs_layout_passes=False)
    matvec = pl.kernel(
        _matvec_body,
        out_type=jax.ShapeDtypeStruct((SC_ROWS,), jnp.float32),
        mesh=mesh,
        compiler_params=params,
        scratch_types=[
            pltpu.VMEM((CHUNK, EMBED_DIM), jnp.float32),
            pltpu.VMEM((CHUNK, EMBED_DIM), jnp.float32),
            pltpu.VMEM((CHUNK,), jnp.float32),
            pltpu.VMEM((EMBED_DIM * LANES,), jnp.float32),
            pltpu.VMEM((80,), jnp.float32),
            pltpu.VMEM((EMBED_DIM * LANES,), jnp.int32),
            pltpu.SemaphoreType.DMA,
            pltpu.SemaphoreType.DMA,
        ],
    )
    pool = pl.kernel(
        _pool_body,
        out_type=jax.ShapeDtypeStruct((BATCH,), jnp.float32),
        mesh=mesh,
        compiler_params=params,
        scratch_types=[
            pltpu.VMEM((TOK_PER_W,), jnp.int32),
            pltpu.VMEM((TOK_PER_W,), jnp.float32),
            pltpu.VMEM((ROWS_PER_W,), jnp.float32),
            pltpu.VMEM((HIST * LANES,), jnp.int32),
            pltpu.SemaphoreType.DMA,
        ],
    )
    return matvec, pool


def _tc_scores_body(table_ref, w_ref, b_ref, out_ref):
    t = table_ref[...]
    w = w_ref[...]
    s = jnp.sum(t * w, axis=1, keepdims=True)
    out_ref[...] = (s + b_ref[0, 0]) * (1.0 / HIST)


def _tc_scores(table, fc_w, fc_b):
    return pl.pallas_call(
        _tc_scores_body,
        grid=(TC_ROWS // TC_BLOCK,),
        in_specs=[
            pl.BlockSpec((TC_BLOCK, EMBED_DIM), lambda i: (i, 0)),
            pl.BlockSpec((1, EMBED_DIM), lambda i: (0, 0)),
            pl.BlockSpec(memory_space=pltpu.SMEM),
        ],
        out_specs=pl.BlockSpec((TC_BLOCK, 1), lambda i: (i, 0)),
        out_shape=jax.ShapeDtypeStruct((TC_ROWS, 1), jnp.float32),
    )(table, fc_w, fc_b.reshape(1, 1))


def kernel(x, table, fc_w, fc_b):
    matvec, pool = _make_kernels()
    aux = jnp.concatenate(
        [jnp.zeros((1,), jnp.float32), fc_w.reshape(-1), fc_b.reshape(-1),
         jnp.zeros((14,), jnp.float32)])
    sc_scores = matvec(table, aux)
    tc_scores = _tc_scores(table, fc_w, fc_b)
    scores = jnp.concatenate([tc_scores.reshape(-1), sc_scores])
    out = pool(x.reshape(-1), scores)
    return out.reshape(BATCH, 1)
